# baseline (speedup 1.0000x reference)
import jax
import jax.numpy as jnp
from jax import lax
from jax.experimental import pallas as pl
from jax.experimental.pallas import tpu as pltpu
from jax.experimental.pallas import tpu_sc as plsc

_NC = 2
_NS = 16
_NW = _NC * _NS
_CHUNK = 128
_BLK = 2048


def _sc_gather_pair(u_tab, m_tab, u2, m2, b):
    d = u_tab.shape[1]
    b_per_w = b // _NW
    n_chunks = b_per_w // _CHUNK
    n_halves = 2
    cph = n_chunks // n_halves
    half = b_per_w // n_halves
    mesh = plsc.VectorSubcoreMesh(core_axis_name="c", subcore_axis_name="s")
    out_t = jax.ShapeDtypeStruct((b, d), jnp.float32)

    @pl.kernel(
        out_type=(out_t, out_t),
        mesh=mesh,
        scratch_types=[
            pltpu.VMEM((n_chunks, _CHUNK), jnp.int32),
            pltpu.VMEM((n_chunks, _CHUNK), jnp.int32),
            pltpu.VMEM((half, d), jnp.float32),
            pltpu.VMEM((half, d), jnp.float32),
            pltpu.SemaphoreType.DMA,
        ],
    )
    def gather_kernel(utab_hbm, mtab_hbm, uidx_hbm, midx_hbm,
                      up_hbm, mp_hbm, uidx_v, midx_v, urows_v, mrows_v, sem):
        wid = lax.axis_index("s") * _NC + lax.axis_index("c")
        base = wid * b_per_w
        row0 = wid * n_chunks
        pltpu.sync_copy(uidx_hbm.at[pl.ds(row0, n_chunks)], uidx_v)
        pltpu.sync_copy(midx_hbm.at[pl.ds(row0, n_chunks)], midx_v)
        for h in range(n_halves):
            copies = []
            for j in range(cph):
                jj = h * cph + j
                sl = pl.ds(j * _CHUNK, _CHUNK)
                copies.append(
                    pltpu.async_copy(utab_hbm.at[uidx_v.at[jj]], urows_v.at[sl], sem))
                copies.append(
                    pltpu.async_copy(mtab_hbm.at[midx_v.at[jj]], mrows_v.at[sl], sem))
            for c in copies:
                c.wait()
            pltpu.sync_copy(urows_v, up_hbm.at[pl.ds(base + h * half, half)])
            pltpu.sync_copy(mrows_v, mp_hbm.at[pl.ds(base + h * half, half)])

    return gather_kernel(u_tab, m_tab, u2, m2)


def _mlp_body(up_ref, mp_ref, su_ref, sm_ref,
              w1a_ref, w1b_ref, b1_ref, w2_ref, b2_ref, out_ref):
    d = up_ref.shape[1] // 2
    up = up_ref[...]
    mp = mp_ref[...]
    ue = jnp.where(su_ref[...] == 0, up[:, :d], up[:, d:])
    me = jnp.where(sm_ref[...] == 0, mp[:, :d], mp[:, d:])
    h = jnp.dot(ue, w1a_ref[...], preferred_element_type=jnp.float32)
    h = h + jnp.dot(me, w1b_ref[...], preferred_element_type=jnp.float32)
    h = h + b1_ref[...]
    h = jnp.maximum(h, 0.0)
    out_ref[...] = (
        jnp.dot(h, w2_ref[...], preferred_element_type=jnp.float32) + b2_ref[...]
    )


def _tc_mlp(up, mp, su, sm, W1, b1, W2, b2):
    b = up.shape[0]
    d = up.shape[1] // 2
    nh = W1.shape[0]
    w1a = W1[:, :d].T
    w1b = W1[:, d:].T
    b1r = b1.reshape(1, nh)
    w2 = W2.T
    b2r = b2.reshape(1, 1)
    grid = (b // _BLK,)
    return pl.pallas_call(
        _mlp_body,
        grid=grid,
        in_specs=[
            pl.BlockSpec((_BLK, 2 * d), lambda i: (i, 0)),
            pl.BlockSpec((_BLK, 2 * d), lambda i: (i, 0)),
            pl.BlockSpec((_BLK, 1), lambda i: (i, 0)),
            pl.BlockSpec((_BLK, 1), lambda i: (i, 0)),
            pl.BlockSpec((d, nh), lambda i: (0, 0)),
            pl.BlockSpec((d, nh), lambda i: (0, 0)),
            pl.BlockSpec((1, nh), lambda i: (0, 0)),
            pl.BlockSpec((nh, 1), lambda i: (0, 0)),
            pl.BlockSpec((1, 1), lambda i: (0, 0)),
        ],
        out_specs=pl.BlockSpec((_BLK, 1), lambda i: (i, 0)),
        out_shape=jax.ShapeDtypeStruct((b, 1), jnp.float32),
    )(up, mp, su, sm, w1a, w1b, b1r, w2, b2r)


def kernel(u, m, u_emb, m_emb, W1, b1, W2, b2):
    b = u.shape[0]
    u_tab = u_emb.reshape(u_emb.shape[0] // 2, 2 * u_emb.shape[1])
    m_tab = m_emb.reshape(m_emb.shape[0] // 2, 2 * m_emb.shape[1])
    u2 = (u >> 1).astype(jnp.int32).reshape(b // _CHUNK, _CHUNK)
    m2 = (m >> 1).astype(jnp.int32).reshape(b // _CHUNK, _CHUNK)
    su = (u & 1).astype(jnp.int32).reshape(b, 1)
    sm = (m & 1).astype(jnp.int32).reshape(b, 1)
    up, mp = _sc_gather_pair(u_tab, m_tab, u2, m2, b)
    return _tc_mlp(up, mp, su, sm, W1, b1, W2, b2)
